# baseline (speedup 1.0000x reference)
import math

import jax
import jax.numpy as jnp
import numpy as np
from jax.experimental import pallas as pl
from jax.experimental.pallas import tpu as pltpu

N = 65536
R = 8
B = 16
LANES = 128
FLAT_ROWS = N * R // LANES
PHS_ROWS = N // LANES
GRID = 8
FB = FLAT_ROWS // GRID
PB = PHS_ROWS // GRID

REF_OSC = np.float32((2.0 * math.pi * 40.0 * 0.001) % (2.0 * math.pi))
STEP = np.float32((math.pi / 4.0) / (N - 1))
C2PI = np.float32(2.0 * math.pi)

_LO = float(REF_OSC)
_HI = float(REF_OSC) + math.pi / 4.0
CENTER = np.float32((_LO + _HI) / 2.0)
CC = np.float32(math.cos((_LO + _HI) / 2.0))
SC = np.float32(math.sin((_LO + _HI) / 2.0))

_sel = (np.arange(128)[None, :] // 8) == np.arange(16)[:, None]
_lin8 = np.linspace(0.0, 2.0 * math.pi, 8)
_lin8_lane = np.tile(np.arange(8), 16)
E01 = _sel.astype(np.float32)
EC = (_sel * np.cos(_lin8)[_lin8_lane][None, :]).astype(np.float32)
ES = (_sel * np.sin(_lin8)[_lin8_lane][None, :]).astype(np.float32)


def _body(spk16_ref, spkd_ref, e01_ref, ec_ref, es_ref,
          pe_ref, flat_ref, phs_ref, lsp_ref, pw_ref):
    g = pl.program_id(0)

    rows_d = jax.lax.broadcasted_iota(jnp.int32, (PB, LANES), 0)
    lane_d = jax.lax.broadcasted_iota(jnp.int32, (PB, LANES), 1)
    i_d = ((g * PB + rows_d) * LANES + lane_d).astype(jnp.float32)
    phid = REF_OSC + i_d * STEP
    phs_ref[...] = phid
    maskd = spkd_ref[...] > 0
    lsp_ref[...] = jnp.where(maskd, phid, -jnp.inf)

    rows_f = jax.lax.broadcasted_iota(jnp.int32, (FB, LANES), 0)
    lane_f = jax.lax.broadcasted_iota(jnp.int32, (FB, LANES), 1)
    i_f = ((g * FB + rows_f) * (LANES // R) + (lane_f >> 3)).astype(jnp.float32)
    k_f = (lane_f & 7).astype(jnp.float32)
    phif = REF_OSC + i_f * STEP
    binf = jnp.floor(phif / C2PI * np.float32(R))
    binf = jnp.clip(binf, 0.0, np.float32(R - 1))

    m16 = (spk16_ref[...] > 0).astype(jnp.float32)
    mrep = jax.lax.dot_general(
        m16, e01_ref[...], (((1,), (0,)), ((), ())),
        preferred_element_type=jnp.float32)
    flat = (np.float32(0.95) * mrep) * (binf == k_f).astype(jnp.float32)
    flat_ref[...] = flat
    pe_ref[...] = jnp.broadcast_to(flat[None], (B, FB, LANES))

    rows_w = jax.lax.broadcasted_iota(jnp.int32, (FB, 16), 0)
    lane_w = jax.lax.broadcasted_iota(jnp.int32, (FB, 16), 1)
    i_w = ((g * FB + rows_w) * 16 + lane_w).astype(jnp.float32)
    x = (REF_OSC + i_w * STEP) - CENTER
    x2 = x * x
    cosx = 1.0 + x2 * (np.float32(-0.5) + x2 * (np.float32(1.0 / 24.0)
                                                + x2 * np.float32(-1.0 / 720.0)))
    sinx = x * (1.0 + x2 * (np.float32(-1.0 / 6.0) + x2 * np.float32(1.0 / 120.0)))
    cphi = CC * cosx - SC * sinx
    sphi = SC * cosx + CC * sinx
    pw = (jax.lax.dot_general(cphi, ec_ref[...], (((1,), (0,)), ((), ())),
                              preferred_element_type=jnp.float32,
                              precision=jax.lax.Precision.HIGHEST)
          + jax.lax.dot_general(sphi, es_ref[...], (((1,), (0,)), ((), ())),
                                preferred_element_type=jnp.float32,
                                precision=jax.lax.Precision.HIGHEST))
    pw_ref[...] = pw


def _run(spk16, spkd, e01, ec, es):
    return pl.pallas_call(
        _body,
        grid=(GRID,),
        in_specs=[
            pl.BlockSpec((FB, 16), lambda g: (g, 0)),
            pl.BlockSpec((PB, LANES), lambda g: (g, 0)),
            pl.BlockSpec((16, LANES), lambda g: (0, 0)),
            pl.BlockSpec((16, LANES), lambda g: (0, 0)),
            pl.BlockSpec((16, LANES), lambda g: (0, 0)),
        ],
        out_specs=[
            pl.BlockSpec((B, FB, LANES), lambda g: (0, g, 0)),
            pl.BlockSpec((FB, LANES), lambda g: (g, 0)),
            pl.BlockSpec((PB, LANES), lambda g: (g, 0)),
            pl.BlockSpec((PB, LANES), lambda g: (g, 0)),
            pl.BlockSpec((FB, LANES), lambda g: (g, 0)),
        ],
        out_shape=[
            jax.ShapeDtypeStruct((B, FLAT_ROWS, LANES), jnp.float32),
            jax.ShapeDtypeStruct((FLAT_ROWS, LANES), jnp.float32),
            jax.ShapeDtypeStruct((PHS_ROWS, LANES), jnp.float32),
            jax.ShapeDtypeStruct((PHS_ROWS, LANES), jnp.float32),
            jax.ShapeDtypeStruct((FLAT_ROWS, LANES), jnp.float32),
        ],
        compiler_params=pltpu.CompilerParams(
            dimension_semantics=("arbitrary",)),
    )(spk16, spkd, e01, ec, es)


def kernel(input_spikes, current_time):
    row0 = input_spikes[0]
    spk16 = row0.reshape(FLAT_ROWS, 16)
    spkd = row0.reshape(PHS_ROWS, LANES)
    pe, flat, phs, lsp, pw = _run(
        spk16, spkd, jnp.asarray(E01), jnp.asarray(EC), jnp.asarray(ES))
    phase_encoded = pe.reshape(B, N * R)
    current_phases = phs.reshape(N)
    phase_bins = flat.reshape(N, R)
    reference_phase = jnp.asarray(REF_OSC, dtype=jnp.float32)
    last_spike_phases = lsp.reshape(N)
    phase_weights = pw.reshape(N, R)
    return (phase_encoded, current_phases, phase_bins, reference_phase,
            last_spike_phases, phase_weights)
